# baseline (speedup 1.0000x reference)
import jax
import jax.numpy as jnp
from jax import lax
from jax.experimental import pallas as pl
from jax.experimental.pallas import tpu as pltpu
from jax.experimental.pallas import tpu_sc as plsc

_NC = 2
_NS = 16
_NW = _NC * _NS
_L = 16
_UNROLL = 16


def _pe_body(x_hbm, emb_hbm, out_hbm, sbuf, lsem, ssem, B, S, D):
    span = S // _NW
    espan = span * D
    rows = span // 2
    chunk = rows * D
    sid = lax.axis_index("s")
    wid = sid * _NC + lax.axis_index("c")
    soff = wid * span
    loads = []
    for b in range(B):
        for q in range(2):
            xoff = (b * S + soff + q * rows) * D
            loads.append(pltpu.async_copy(
                x_hbm.at[pl.ds(xoff, chunk)], sbuf.at[sid], lsem))
    for cp in loads:
        cp.wait()
    stores = []
    for b in range(B):
        for q in range(2):
            xoff = (b * S + soff + q * rows) * D
            stores.append(pltpu.async_copy(
                sbuf.at[sid], out_hbm.at[pl.ds(xoff, chunk)], ssem))
    for cp in stores:
        cp.wait()


def kernel(x, emb_table):
    B, S, D = x.shape
    span = S // _NW
    mesh = plsc.VectorSubcoreMesh(
        core_axis_name="c", subcore_axis_name="s",
        num_cores=_NC, num_subcores=_NS,
    )
    f = pl.kernel(
        lambda xh, eh, oh, sb, ls, ss: _pe_body(
            xh, eh, oh, sb, ls, ss, B, S, D),
        out_type=jax.ShapeDtypeStruct((B * S * D,), x.dtype),
        mesh=mesh,
        scratch_types=[
            pltpu.VMEM_SHARED((_NS, span // 2 * D), jnp.float32),
            pltpu.SemaphoreType.DMA,
            pltpu.SemaphoreType.DMA,
        ],
    )
    out = f(x.reshape(-1), emb_table.reshape(-1))
    return out.reshape(B, S, D)


# speedup vs baseline: 5.2082x; 5.2082x over previous
import jax
import jax.numpy as jnp
from jax.experimental import pallas as pl


def _add_body(x_ref, e_ref, o_ref):
    o_ref[...] = x_ref[...] + e_ref[...]


def kernel(x, emb_table):
    B, S, D = x.shape
    BLK = 512
    return pl.pallas_call(
        _add_body,
        grid=(S // BLK,),
        in_specs=[
            pl.BlockSpec((B, BLK, D), lambda i: (0, i, 0)),
            pl.BlockSpec((BLK, D), lambda i: (i, 0)),
        ],
        out_specs=pl.BlockSpec((B, BLK, D), lambda i: (0, i, 0)),
        out_shape=jax.ShapeDtypeStruct((B, S, D), x.dtype),
    )(x, emb_table)
